# baseline (speedup 1.0000x reference)
import functools

import jax
import jax.numpy as jnp
from jax import lax
from jax.experimental import pallas as pl
from jax.experimental.pallas import tpu as pltpu
from jax.experimental.pallas import tpu_sc as plsc

B = 4
R = 8192
C = 4096
TOTAL = B * C
LANES = 16

_info = plsc.get_sparse_core_info()
NC = _info.num_cores
NS = _info.num_subcores
NW = NC * NS
PER_W = TOTAL // NW
CHUNK = 128
NCHUNK = PER_W // CHUNK

_mesh = plsc.VectorSubcoreMesh(core_axis_name="c", subcore_axis_name="s")


@functools.partial(
    pl.kernel,
    mesh=_mesh,
    out_type=jax.ShapeDtypeStruct((TOTAL,), jnp.float32),
    scratch_types=[
        pltpu.VMEM((PER_W,), jnp.int32),
        pltpu.VMEM((NCHUNK, CHUNK), jnp.int32),
        pltpu.VMEM((NCHUNK, CHUNK), jnp.float32),
        pltpu.SemaphoreType.DMA,
    ],
)
def _gather_kernel(x_hbm, idx_hbm, out_hbm, idx_v, fidx_v, vals_v, sem):
    wid = lax.axis_index("s") * NC + lax.axis_index("c")
    base = wid * PER_W
    b = base // C
    boff = b * (R * C)
    c0 = base - b * C

    pltpu.sync_copy(idx_hbm.at[pl.ds(base, PER_W)], idx_v)

    lane = lax.iota(jnp.int32, LANES)
    for k in range(NCHUNK):
        for j in range(CHUNK // LANES):
            off = k * CHUNK + j * LANES
            rows = idx_v[pl.ds(off, LANES)]
            fidx_v[k, pl.ds(j * LANES, LANES)] = (
                rows * C + (boff + c0 + off) + lane
            )

    copies = [
        pltpu.async_copy(x_hbm.at[fidx_v.at[k]], vals_v.at[k], sem)
        for k in range(NCHUNK)
    ]
    for cp in copies:
        cp.wait()

    for k in range(NCHUNK):
        pltpu.sync_copy(vals_v.at[k], out_hbm.at[pl.ds(base + k * CHUNK, CHUNK)])


def kernel(x, index):
    x_flat = x.reshape(B * R * C)
    idx_flat = index.reshape(TOTAL)
    out = _gather_kernel(x_flat, idx_flat)
    return out.reshape(B, 1, C)


# speedup vs baseline: 14.3025x; 14.3025x over previous
import functools

import jax
import jax.numpy as jnp
from jax import lax
from jax.experimental import pallas as pl
from jax.experimental.pallas import tpu as pltpu
from jax.experimental.pallas import tpu_sc as plsc

B = 4
R = 8192
C = 4096
TOTAL = B * C
LANES = 16

_info = plsc.get_sparse_core_info()
NC = _info.num_cores
NS = _info.num_subcores
NW = NC * NS
PER_W = TOTAL // NW
WIN = 128
NGROUP = PER_W // WIN

_mesh = plsc.VectorSubcoreMesh(core_axis_name="c", subcore_axis_name="s")


@functools.partial(
    pl.kernel,
    mesh=_mesh,
    out_type=jax.ShapeDtypeStruct((TOTAL,), jnp.float32),
    scratch_types=[
        pltpu.VMEM((PER_W,), jnp.int32),
        pltpu.VMEM((NGROUP, WIN, WIN), jnp.float32),
        pltpu.VMEM((PER_W,), jnp.float32),
        pltpu.SemaphoreType.DMA,
    ],
    compiler_params=pltpu.CompilerParams(needs_layout_passes=False),
)
def _gather_kernel(x_hbm, idx_hbm, out_hbm, idx_v, vals_v, res_v, sem):
    wid = lax.axis_index("s") * NC + lax.axis_index("c")
    base = wid * PER_W
    b = base // C
    c0 = base - b * C

    pltpu.sync_copy(idx_hbm.at[pl.ds(base, PER_W)], idx_v)

    for j in range(PER_W // LANES):
        sl = pl.ds(j * LANES, LANES)
        idx_v[sl] = idx_v[sl] + (b * R)

    copies = []
    for g in range(NGROUP):
        rows = idx_v.at[pl.ds(g * WIN, WIN)]
        cw = c0 + g * WIN
        copies.append(
            pltpu.async_copy(
                x_hbm.at[rows, pl.ds(cw, WIN)], vals_v.at[g], sem
            )
        )
    for cp in copies:
        cp.wait()

    lane = lax.iota(jnp.int32, LANES)

    for g in range(NGROUP):
        for j in range(WIN // LANES):
            sel = lane + (j * LANES)
            res_v[pl.ds(g * WIN + j * LANES, LANES)] = plsc.load_gather(
                vals_v.at[g], [sel, sel]
            )

    pltpu.sync_copy(res_v, out_hbm.at[pl.ds(base, PER_W)])


def kernel(x, index):
    x2d = x.reshape(B * R, C)
    idx_flat = index.reshape(TOTAL)
    out = _gather_kernel(x2d, idx_flat)
    return out.reshape(B, 1, C)
